# baseline (speedup 1.0000x reference)
import functools

import jax
import jax.numpy as jnp
from jax import lax
from jax.experimental import pallas as pl
from jax.experimental.pallas import tpu as pltpu
from jax.experimental.pallas import tpu_sc as plsc

N_NODES = 10000
E_EDGES = 320000
NC = 2
NS = 16
NW = NC * NS
NPW = 313
N_PAD = NPW * NW
EPW = E_EDGES // NW
GC = 400
SCAN_CHUNK = 8000
SUB = 512
F = 64

_mesh = plsc.VectorSubcoreMesh(core_axis_name="c", subcore_axis_name="s")
_sc_params = pltpu.CompilerParams(
    use_tc_tiling_on_sc=False, needs_layout_passes=False
)


def _wid():
    return lax.axis_index("s") * NC + lax.axis_index("c")


@functools.partial(
    pl.kernel,
    out_type=jax.ShapeDtypeStruct((E_EDGES, F), jnp.float32),
    mesh=_mesh,
    scratch_types=[
        pltpu.VMEM((GC,), jnp.int32),
        pltpu.VMEM((GC,), jnp.int32),
        pltpu.VMEM((GC, F), jnp.float32),
        pltpu.VMEM((GC, F), jnp.float32),
        pltpu.SemaphoreType.DMA,
    ],
    compiler_params=_sc_params,
)
def _edge_gather(a_hbm, b_hbm, src_hbm, dst_hbm, t_hbm,
                 dstv, srcv, arows, brows, sem):
    ebase = _wid() * EPW

    def chunk(c, _):
        base = ebase + c * GC
        pltpu.sync_copy(dst_hbm.at[pl.ds(base, GC)], dstv)
        pltpu.sync_copy(src_hbm.at[pl.ds(base, GC)], srcv)
        pltpu.async_copy(a_hbm.at[dstv], arows, sem).wait()
        pltpu.async_copy(b_hbm.at[srcv], brows, sem).wait()

        def row(r, _):
            for k in range(F // 16):
                s = pl.ds(k * 16, 16)
                arows[r, s] = jnp.maximum(arows[r, s] + brows[r, s], 0.0)
            return 0

        lax.fori_loop(0, GC, row, 0)
        pltpu.sync_copy(arows, t_hbm.at[pl.ds(base, GC)])
        return 0

    lax.fori_loop(0, EPW // GC, chunk, 0)


@functools.partial(
    pl.kernel,
    out_type=jax.ShapeDtypeStruct((N_PAD, F), jnp.float32),
    mesh=_mesh,
    scratch_types=[
        pltpu.VMEM((SCAN_CHUNK,), jnp.int32),
        pltpu.VMEM((SCAN_CHUNK + SUB,), jnp.int32),
        pltpu.VMEM((SCAN_CHUNK + SUB,), jnp.int32),
        pltpu.VMEM((SUB, F), jnp.float32),
        pltpu.VMEM((NPW + 1, F), jnp.float32),
        pltpu.SemaphoreType.DMA,
    ],
    compiler_params=_sc_params,
)
def _seg_max(m_hbm, dst_hbm, h_hbm, idxv, eidv, dstlv, mrows, acc, sem):
    wid = _wid()
    lo = wid * NPW
    hi = lo + NPW
    neg_inf = jnp.full((16,), -jnp.inf, dtype=jnp.float32)
    zeros16 = jnp.zeros((16,), jnp.int32)
    lanes = lax.iota(jnp.int32, 16)

    def init_acc(r, _):
        for k in range(F // 16):
            acc[r, pl.ds(k * 16, 16)] = neg_inf
        return 0

    lax.fori_loop(0, NPW + 1, init_acc, 0)

    def init_eid(v, _):
        eidv[pl.ds(v * 16, 16)] = zeros16
        return 0

    lax.fori_loop(0, (SCAN_CHUNK + SUB) // 16, init_eid, 0)

    def chunk(c, _):
        cbase = c * SCAN_CHUNK
        pltpu.sync_copy(dst_hbm.at[pl.ds(cbase, SCAN_CHUNK)], idxv)

        def scan_v(v, cur):
            d16 = idxv[pl.ds(v * 16, 16)]
            msk = jnp.logical_and(d16 >= lo, d16 < hi)
            mi = msk.astype(jnp.int32)
            inc = jnp.cumsum(mi)
            pos = cur + inc - 1
            eid16 = cbase + v * 16 + lanes
            plsc.store_scatter(eidv, [pos], eid16, mask=msk)
            plsc.store_scatter(dstlv, [pos], d16 - lo, mask=msk)
            return cur + jnp.sum(mi)

        nmatch = lax.fori_loop(0, SCAN_CHUNK // 16, scan_v, 0)

        def sub(b, _):
            rbase = b * SUB
            pltpu.async_copy(m_hbm.at[eidv.at[pl.ds(rbase, SUB)]],
                             mrows, sem).wait()
            cnt = jnp.minimum(nmatch - rbase, SUB)

            def group(g, _):
                d16 = dstlv[pl.ds(rbase + g * 16, 16)]
                valid = lanes < (cnt - g * 16)
                d16s = jnp.where(valid, d16, NPW)
                for lane in range(16):
                    dloc = d16s[lane]
                    r = g * 16 + lane
                    for k in range(F // 16):
                        s = pl.ds(k * 16, 16)
                        acc[dloc, s] = jnp.maximum(acc[dloc, s], mrows[r, s])
                return 0

            lax.fori_loop(0, (cnt + 15) // 16, group, 0)
            return 0

        lax.fori_loop(0, (nmatch + SUB - 1) // SUB, sub, 0)
        return 0

    lax.fori_loop(0, E_EDGES // SCAN_CHUNK, chunk, 0)
    pltpu.sync_copy(acc.at[pl.ds(0, NPW)], h_hbm.at[pl.ds(lo, NPW)])


def _ab_kernel(h_ref, wd_ref, wb_ref, ba_ref, oa_ref, ob_ref):
    h = h_ref[...]
    oa_ref[...] = h @ wd_ref[...] + ba_ref[...]
    ob_ref[...] = h @ wb_ref[...]


def _ab_from_agg_kernel(hraw_ref, bprev_ref, wd_ref, wb_ref, ba_ref,
                        oa_ref, ob_ref):
    hr = hraw_ref[...]
    h = jnp.maximum(jnp.where(jnp.isfinite(hr), hr + bprev_ref[...], 0.0), 0.0)
    oa_ref[...] = h @ wd_ref[...] + ba_ref[...]
    ob_ref[...] = h @ wb_ref[...]


def _mm_kernel(t_ref, w_ref, o_ref):
    o_ref[...] = t_ref[...] @ w_ref[...]


def _head_kernel(hraw_ref, bprev_ref, wl_ref, bl_ref, o_ref):
    hr = hraw_ref[...]
    h = jnp.maximum(jnp.where(jnp.isfinite(hr), hr + bprev_ref[...], 0.0), 0.0)
    o_ref[...] = h @ wl_ref[...] + bl_ref[...]


def _tc_ab(h, wd, wbot, ba):
    n = h.shape[0]
    return pl.pallas_call(
        _ab_kernel,
        out_shape=(jax.ShapeDtypeStruct((n, F), jnp.float32),
                   jax.ShapeDtypeStruct((n, F), jnp.float32)),
    )(h, wd, wbot, ba.reshape(1, F))


def _tc_ab_from_agg(hraw, bprev, wd, wbot, ba):
    n = hraw.shape[0]
    return pl.pallas_call(
        _ab_from_agg_kernel,
        out_shape=(jax.ShapeDtypeStruct((n, F), jnp.float32),
                   jax.ShapeDtypeStruct((n, F), jnp.float32)),
    )(hraw, bprev.reshape(1, F), wd, wbot, ba.reshape(1, F))


def _tc_mm(t, w):
    blk = 8000
    return pl.pallas_call(
        _mm_kernel,
        grid=(E_EDGES // blk,),
        in_specs=[pl.BlockSpec((blk, F), lambda i: (i, 0)),
                  pl.BlockSpec((F, F), lambda i: (0, 0))],
        out_specs=pl.BlockSpec((blk, F), lambda i: (i, 0)),
        out_shape=jax.ShapeDtypeStruct((E_EDGES, F), jnp.float32),
    )(t, w)


def _tc_head(hraw, bprev, wl, bl):
    n = hraw.shape[0]
    return pl.pallas_call(
        _head_kernel,
        out_shape=jax.ShapeDtypeStruct((n, 1), jnp.float32),
    )(hraw, bprev.reshape(1, F), wl, bl.reshape(1, 1))


def kernel(x, edge_index, W1a, b1a, W2a, b2a, W1b, b1b, W2b, b2b, Wl, bl):
    src = edge_index[0]
    dst = edge_index[1]

    d = x.shape[1]
    wd1, wbot1 = W1a[:d] - W1a[d:], W1a[d:]
    wd2, wbot2 = W1b[:F] - W1b[F:], W1b[F:]

    a1, b1 = _tc_ab(x, wd1, wbot1, b1a)
    t1 = _edge_gather(a1, b1, src, dst)
    m1 = _tc_mm(t1, W2a)
    h1 = _seg_max(m1, dst)

    a2, b2 = _tc_ab_from_agg(h1, b2a, wd2, wbot2, b1b)
    t2 = _edge_gather(a2, b2, src, dst)
    m2 = _tc_mm(t2, W2b)
    h2 = _seg_max(m2, dst)

    out = _tc_head(h2, b2b, Wl, bl)
    return out[:N_NODES, 0]


# speedup vs baseline: 5.8595x; 5.8595x over previous
import functools

import jax
import jax.numpy as jnp
from jax import lax
from jax.experimental import pallas as pl
from jax.experimental.pallas import tpu as pltpu
from jax.experimental.pallas import tpu_sc as plsc

N_NODES = 10000
E_EDGES = 320000
E_PAD = 328000
NC = 2
NS = 16
NW = NC * NS
NPW = 313
N_PAD = NPW * NW
EPW = E_EDGES // NW
GC = 400
SUB = 512
F = 64
INV_NPW = 1.0 / NPW

_mesh = plsc.VectorSubcoreMesh(core_axis_name="c", subcore_axis_name="s")
_sc_params = pltpu.CompilerParams(
    use_tc_tiling_on_sc=False, needs_layout_passes=False
)


def _wid():
    return lax.axis_index("s") * NC + lax.axis_index("c")


def _al8(i):
    return pl.multiple_of(i, 8)


def _owner(d16):
    o = (d16.astype(jnp.float32) * INV_NPW).astype(jnp.int32)
    o = jnp.where(d16 < o * NPW, o - 1, o)
    o = jnp.where(d16 >= (o + 1) * NPW, o + 1, o)
    return o


@functools.partial(
    pl.kernel,
    out_type=jax.ShapeDtypeStruct((NW, NW), jnp.int32),
    mesh=_mesh,
    scratch_types=[
        pltpu.VMEM((GC,), jnp.int32),
        pltpu.VMEM((NW,), jnp.int32),
    ],
    compiler_params=_sc_params,
)
def _hist(dst_hbm, cnt_hbm, dstv, hist):
    wid = _wid()
    ebase = wid * EPW
    hist[pl.ds(0, 16)] = jnp.zeros((16,), jnp.int32)
    hist[pl.ds(16, 16)] = jnp.zeros((16,), jnp.int32)

    def chunk(c, _):
        pltpu.sync_copy(dst_hbm.at[pl.ds(_al8(ebase + c * GC), GC)], dstv)

        def vec(v, _):
            o16 = _owner(dstv[pl.ds(v * 16, 16)])
            cnt1, lastm = plsc.scan_count(o16)
            plsc.addupdate_scatter(hist, [o16], cnt1, mask=lastm)
            return 0

        lax.fori_loop(0, GC // 16, vec, 0)
        return 0

    lax.fori_loop(0, EPW // GC, chunk, 0)
    pltpu.sync_copy(hist, cnt_hbm.at[wid])


@functools.partial(
    pl.kernel,
    out_type=(jax.ShapeDtypeStruct((E_EDGES,), jnp.int32),
              jax.ShapeDtypeStruct((E_PAD,), jnp.int32)),
    mesh=_mesh,
    scratch_types=[
        pltpu.VMEM((NW, NW), jnp.int32),
        pltpu.VMEM((GC,), jnp.int32),
        pltpu.VMEM((GC // 80, 80), jnp.int32),
        pltpu.VMEM((GC // 80, 80), jnp.int32),
        pltpu.VMEM((NW,), jnp.int32),
        pltpu.SemaphoreType.DMA,
    ],
    compiler_params=_sc_params,
)
def _binner(dst_hbm, cnt_hbm, slot_hbm, bdstl_hbm,
            cntv, dstv, slotbuf, dstlbuf, cursors, sem):
    wid = _wid()
    ebase = wid * EPW
    lanes = lax.iota(jnp.int32, 16)
    pltpu.sync_copy(cnt_hbm, cntv)

    t0 = jnp.zeros((16,), jnp.int32)
    t1 = jnp.zeros((16,), jnp.int32)
    p0 = jnp.zeros((16,), jnp.int32)
    p1 = jnp.zeros((16,), jnp.int32)
    for w in range(NW):
        r0 = cntv[w, pl.ds(0, 16)]
        r1 = cntv[w, pl.ds(16, 16)]
        t0 = t0 + r0
        t1 = t1 + r1
        keep = jnp.where(jnp.full((16,), w, jnp.int32) < wid, 1, 0)
        p0 = p0 + r0 * keep
        p1 = p1 + r1 * keep
    r80 = (t0 + 7) & ~7
    r81 = (t1 + 7) & ~7
    ex0 = jnp.cumsum(r80) - r80
    ex1 = jnp.cumsum(r81) - r81 + jnp.sum(r80)
    cursors[pl.ds(0, 16)] = ex0 + p0
    cursors[pl.ds(16, 16)] = ex1 + p1

    def chunk(c, _):
        base = _al8(ebase + c * GC)
        pltpu.sync_copy(dst_hbm.at[pl.ds(base, GC)], dstv)

        def vec(v, _):
            r = v // 5
            cc = (v % 5) * 16
            s = pl.ds(cc, 16)
            d16 = dstv[pl.ds(v * 16, 16)]
            o16 = _owner(d16)
            cnt1, lastm = plsc.scan_count(o16)
            old = plsc.load_gather(cursors, [o16])
            slotbuf[r, s] = old + cnt1 - 1
            dstlbuf[r, s] = d16 - o16 * NPW
            plsc.addupdate_scatter(cursors, [o16], cnt1, mask=lastm)
            return 0

        lax.fori_loop(0, GC // 16, vec, 0)
        for j in range(GC // 80):
            pltpu.sync_copy(slotbuf.at[j],
                            slot_hbm.at[pl.ds(base + j * 80, 80)])
            pltpu.async_copy(dstlbuf.at[j], bdstl_hbm.at[slotbuf.at[j]],
                             sem).wait()
        return 0

    lax.fori_loop(0, EPW // GC, chunk, 0)


@functools.partial(
    pl.kernel,
    out_type=jax.ShapeDtypeStruct((E_PAD, F), jnp.float32),
    mesh=_mesh,
    scratch_types=[
        pltpu.VMEM((GC,), jnp.int32),
        pltpu.VMEM((GC,), jnp.int32),
        pltpu.VMEM((GC // 80, 80), jnp.int32),
        pltpu.VMEM((GC, F), jnp.float32),
        pltpu.VMEM((GC, F), jnp.float32),
        pltpu.SemaphoreType.DMA,
    ],
    compiler_params=_sc_params,
)
def _edge_gather(a_hbm, b_hbm, src_hbm, dst_hbm, slot_hbm, t_hbm,
                 dstv, srcv, slotv, arows, brows, sem):
    ebase = _wid() * EPW

    def chunk(c, _):
        base = _al8(ebase + c * GC)
        pltpu.sync_copy(dst_hbm.at[pl.ds(base, GC)], dstv)
        pltpu.sync_copy(src_hbm.at[pl.ds(base, GC)], srcv)
        for j in range(GC // 80):
            pltpu.sync_copy(slot_hbm.at[pl.ds(base + j * 80, 80)],
                            slotv.at[j])
        pltpu.async_copy(a_hbm.at[dstv], arows, sem).wait()
        pltpu.async_copy(b_hbm.at[srcv], brows, sem).wait()

        def row(r, _):
            for k in range(F // 16):
                s = pl.ds(k * 16, 16)
                arows[r, s] = jnp.maximum(arows[r, s] + brows[r, s], 0.0)
            return 0

        lax.fori_loop(0, GC, row, 0)
        for j in range(GC // 80):
            pltpu.async_copy(arows.at[pl.ds(j * 80, 80)],
                             t_hbm.at[slotv.at[j]], sem).wait()
        return 0

    lax.fori_loop(0, EPW // GC, chunk, 0)


@functools.partial(
    pl.kernel,
    out_type=jax.ShapeDtypeStruct((N_PAD, F), jnp.float32),
    mesh=_mesh,
    scratch_types=[
        pltpu.VMEM((NW, NW), jnp.int32),
        pltpu.VMEM((SUB,), jnp.int32),
        pltpu.VMEM((SUB, F), jnp.float32),
        pltpu.VMEM((NPW + 1, F), jnp.float32),
        pltpu.SemaphoreType.DMA,
    ],
    compiler_params=_sc_params,
)
def _seg_max(m_hbm, bdstl_hbm, cnt_hbm, h_hbm, cntv, dstlv, mrows, acc, sem):
    wid = _wid()
    lanes = lax.iota(jnp.int32, 16)
    neg_inf = jnp.full((16,), -jnp.inf, dtype=jnp.float32)
    pltpu.sync_copy(cnt_hbm, cntv)

    t0 = jnp.zeros((16,), jnp.int32)
    t1 = jnp.zeros((16,), jnp.int32)
    for w in range(NW):
        t0 = t0 + cntv[w, pl.ds(0, 16)]
        t1 = t1 + cntv[w, pl.ds(16, 16)]
    r80 = (t0 + 7) & ~7
    r81 = (t1 + 7) & ~7
    ex0 = jnp.cumsum(r80) - r80
    ex1 = jnp.cumsum(r81) - r81 + jnp.sum(r80)
    m0 = jnp.where(lanes == wid, 1, 0)
    m1 = jnp.where(lanes == wid - 16, 1, 0)
    bstart = jnp.sum(ex0 * m0) + jnp.sum(ex1 * m1)
    total = jnp.sum(t0 * m0) + jnp.sum(t1 * m1)

    def init_acc(r, _):
        for k in range(F // 16):
            acc[r, pl.ds(k * 16, 16)] = neg_inf
        return 0

    lax.fori_loop(0, NPW + 1, init_acc, 0)

    def chunk(c, _):
        off = _al8(bstart + c * SUB)
        pltpu.sync_copy(bdstl_hbm.at[pl.ds(off, SUB)], dstlv)
        pltpu.sync_copy(m_hbm.at[pl.ds(off, SUB)], mrows)
        cnt = jnp.minimum(total - c * SUB, SUB)

        def group(g, _):
            d16 = dstlv[pl.ds(g * 16, 16)]
            valid = lanes < (cnt - g * 16)
            d16s = jnp.where(valid, d16, NPW)
            for lane in range(16):
                dloc = d16s[lane]
                r = g * 16 + lane
                for k in range(F // 16):
                    s = pl.ds(k * 16, 16)
                    acc[dloc, s] = jnp.maximum(acc[dloc, s], mrows[r, s])
            return 0

        lax.fori_loop(0, (cnt + 15) // 16, group, 0)
        return 0

    lax.fori_loop(0, (total + SUB - 1) // SUB, chunk, 0)
    pltpu.sync_copy(acc.at[pl.ds(0, NPW)], h_hbm.at[pl.ds(wid * NPW, NPW)])


def _ab_kernel(h_ref, wd_ref, wb_ref, ba_ref, oa_ref, ob_ref):
    h = h_ref[...]
    oa_ref[...] = h @ wd_ref[...] + ba_ref[...]
    ob_ref[...] = h @ wb_ref[...]


def _ab_from_agg_kernel(hraw_ref, bprev_ref, wd_ref, wb_ref, ba_ref,
                        oa_ref, ob_ref):
    hr = hraw_ref[...]
    h = jnp.maximum(jnp.where(jnp.isfinite(hr), hr + bprev_ref[...], 0.0), 0.0)
    oa_ref[...] = h @ wd_ref[...] + ba_ref[...]
    ob_ref[...] = h @ wb_ref[...]


def _mm_kernel(t_ref, w_ref, o_ref):
    o_ref[...] = t_ref[...] @ w_ref[...]


def _head_kernel(hraw_ref, bprev_ref, wl_ref, bl_ref, o_ref):
    hr = hraw_ref[...]
    h = jnp.maximum(jnp.where(jnp.isfinite(hr), hr + bprev_ref[...], 0.0), 0.0)
    o_ref[...] = h @ wl_ref[...] + bl_ref[...]


def _tc_ab(h, wd, wbot, ba):
    n = h.shape[0]
    return pl.pallas_call(
        _ab_kernel,
        out_shape=(jax.ShapeDtypeStruct((n, F), jnp.float32),
                   jax.ShapeDtypeStruct((n, F), jnp.float32)),
    )(h, wd, wbot, ba.reshape(1, F))


def _tc_ab_from_agg(hraw, bprev, wd, wbot, ba):
    n = hraw.shape[0]
    return pl.pallas_call(
        _ab_from_agg_kernel,
        out_shape=(jax.ShapeDtypeStruct((n, F), jnp.float32),
                   jax.ShapeDtypeStruct((n, F), jnp.float32)),
    )(hraw, bprev.reshape(1, F), wd, wbot, ba.reshape(1, F))


def _tc_mm(t, w):
    blk = 8000
    return pl.pallas_call(
        _mm_kernel,
        grid=(E_PAD // blk,),
        in_specs=[pl.BlockSpec((blk, F), lambda i: (i, 0)),
                  pl.BlockSpec((F, F), lambda i: (0, 0))],
        out_specs=pl.BlockSpec((blk, F), lambda i: (i, 0)),
        out_shape=jax.ShapeDtypeStruct((E_PAD, F), jnp.float32),
    )(t, w)


def _tc_head(hraw, bprev, wl, bl):
    n = hraw.shape[0]
    return pl.pallas_call(
        _head_kernel,
        out_shape=jax.ShapeDtypeStruct((n, 1), jnp.float32),
    )(hraw, bprev.reshape(1, F), wl, bl.reshape(1, 1))


def kernel(x, edge_index, W1a, b1a, W2a, b2a, W1b, b1b, W2b, b2b, Wl, bl):
    src = edge_index[0]
    dst = edge_index[1]

    d = x.shape[1]
    wd1, wbot1 = W1a[:d] - W1a[d:], W1a[d:]
    wd2, wbot2 = W1b[:F] - W1b[F:], W1b[F:]

    counts = _hist(dst)
    slots, bdstl = _binner(dst, counts)

    a1, b1 = _tc_ab(x, wd1, wbot1, b1a)
    t1 = _edge_gather(a1, b1, src, dst, slots)
    m1 = _tc_mm(t1, W2a)
    h1 = _seg_max(m1, bdstl, counts)

    a2, b2 = _tc_ab_from_agg(h1, b2a, wd2, wbot2, b1b)
    t2 = _edge_gather(a2, b2, src, dst, slots)
    m2 = _tc_mm(t2, W2b)
    h2 = _seg_max(m2, bdstl, counts)

    out = _tc_head(h2, b2b, Wl, bl)
    return out[:N_NODES, 0]


# speedup vs baseline: 6.3755x; 1.0880x over previous
import functools

import jax
import jax.numpy as jnp
from jax import lax
from jax.experimental import pallas as pl
from jax.experimental.pallas import tpu as pltpu
from jax.experimental.pallas import tpu_sc as plsc

N_NODES = 10000
E_EDGES = 320000
E_PAD = 328000
NC = 2
NS = 16
NW = NC * NS
NPW = 313
N_PAD = NPW * NW
EPW = E_EDGES // NW
GC = 400
SUB = 512
F = 64
INV_NPW = 1.0 / NPW

_mesh = plsc.VectorSubcoreMesh(core_axis_name="c", subcore_axis_name="s")
_sc_params = pltpu.CompilerParams(
    use_tc_tiling_on_sc=False, needs_layout_passes=False
)


def _wid():
    return lax.axis_index("s") * NC + lax.axis_index("c")


def _al8(i):
    return pl.multiple_of(i, 8)


def _owner(d16):
    o = (d16.astype(jnp.float32) * INV_NPW).astype(jnp.int32)
    o = jnp.where(d16 < o * NPW, o - 1, o)
    o = jnp.where(d16 >= (o + 1) * NPW, o + 1, o)
    return o


@functools.partial(
    pl.kernel,
    out_type=jax.ShapeDtypeStruct((NW, NW), jnp.int32),
    mesh=_mesh,
    scratch_types=[
        pltpu.VMEM((GC,), jnp.int32),
        pltpu.VMEM((NW,), jnp.int32),
    ],
    compiler_params=_sc_params,
)
def _hist(dst_hbm, cnt_hbm, dstv, hist):
    wid = _wid()
    ebase = wid * EPW
    hist[pl.ds(0, 16)] = jnp.zeros((16,), jnp.int32)
    hist[pl.ds(16, 16)] = jnp.zeros((16,), jnp.int32)

    def chunk(c, _):
        pltpu.sync_copy(dst_hbm.at[pl.ds(_al8(ebase + c * GC), GC)], dstv)

        def vec(v, _):
            o16 = _owner(dstv[pl.ds(v * 16, 16)])
            cnt1, lastm = plsc.scan_count(o16)
            plsc.addupdate_scatter(hist, [o16], cnt1, mask=lastm)
            return 0

        lax.fori_loop(0, GC // 16, vec, 0)
        return 0

    lax.fori_loop(0, EPW // GC, chunk, 0)
    pltpu.sync_copy(hist, cnt_hbm.at[wid])


@functools.partial(
    pl.kernel,
    out_type=(jax.ShapeDtypeStruct((E_EDGES,), jnp.int32),
              jax.ShapeDtypeStruct((E_PAD,), jnp.int32)),
    mesh=_mesh,
    scratch_types=[
        pltpu.VMEM((NW, NW), jnp.int32),
        pltpu.VMEM((GC,), jnp.int32),
        pltpu.VMEM((GC // 80, 80), jnp.int32),
        pltpu.VMEM((GC,), jnp.int32),
        pltpu.VMEM((GC // 80, 80), jnp.int32),
        pltpu.VMEM((NW,), jnp.int32),
        pltpu.SemaphoreType.DMA,
    ],
    compiler_params=_sc_params,
)
def _binner(dst_hbm, cnt_hbm, slot_hbm, bdstl_hbm,
            cntv, dstv, slotbuf, slotlin, dstlbuf, cursors, sem):
    wid = _wid()
    ebase = wid * EPW
    lanes = lax.iota(jnp.int32, 16)
    pltpu.sync_copy(cnt_hbm, cntv)

    t0 = jnp.zeros((16,), jnp.int32)
    t1 = jnp.zeros((16,), jnp.int32)
    p0 = jnp.zeros((16,), jnp.int32)
    p1 = jnp.zeros((16,), jnp.int32)
    for w in range(NW):
        r0 = cntv[w, pl.ds(0, 16)]
        r1 = cntv[w, pl.ds(16, 16)]
        t0 = t0 + r0
        t1 = t1 + r1
        keep = jnp.where(jnp.full((16,), w, jnp.int32) < wid, 1, 0)
        p0 = p0 + r0 * keep
        p1 = p1 + r1 * keep
    r80 = (t0 + 7) & ~7
    r81 = (t1 + 7) & ~7
    ex0 = jnp.cumsum(r80) - r80
    ex1 = jnp.cumsum(r81) - r81 + jnp.sum(r80)
    cursors[pl.ds(0, 16)] = ex0 + p0
    cursors[pl.ds(16, 16)] = ex1 + p1

    def chunk(c, _):
        base = _al8(ebase + c * GC)
        pltpu.sync_copy(dst_hbm.at[pl.ds(base, GC)], dstv)

        def vec(v, _):
            r = v // 5
            cc = (v % 5) * 16
            s = pl.ds(cc, 16)
            d16 = dstv[pl.ds(v * 16, 16)]
            o16 = _owner(d16)
            cnt1, lastm = plsc.scan_count(o16)
            old = plsc.load_gather(cursors, [o16])
            slot16 = old + cnt1 - 1
            slotbuf[r, s] = slot16
            slotlin[pl.ds(v * 16, 16)] = slot16
            dstlbuf[r, s] = d16 - o16 * NPW
            plsc.addupdate_scatter(cursors, [o16], cnt1, mask=lastm)
            return 0

        lax.fori_loop(0, GC // 16, vec, 0)
        cps = [pltpu.async_copy(slotlin, slot_hbm.at[pl.ds(base, GC)], sem)]
        for j in range(GC // 80):
            cps.append(pltpu.async_copy(dstlbuf.at[j],
                                        bdstl_hbm.at[slotbuf.at[j]], sem))
        for cp in cps:
            cp.wait()
        return 0

    lax.fori_loop(0, EPW // GC, chunk, 0)


@functools.partial(
    pl.kernel,
    out_type=jax.ShapeDtypeStruct((E_PAD, F), jnp.float32),
    mesh=_mesh,
    scratch_types=[
        pltpu.VMEM((GC,), jnp.int32),
        pltpu.VMEM((GC,), jnp.int32),
        pltpu.VMEM((GC // 80, 80), jnp.int32),
        pltpu.VMEM((GC, F), jnp.float32),
        pltpu.VMEM((GC, F), jnp.float32),
        pltpu.SemaphoreType.DMA,
    ],
    compiler_params=_sc_params,
)
def _edge_gather(a_hbm, b_hbm, src_hbm, dst_hbm, slot_hbm, t_hbm,
                 dstv, srcv, slotv, arows, brows, sem):
    ebase = _wid() * EPW

    def chunk(c, _):
        base = _al8(ebase + c * GC)
        cps = [pltpu.async_copy(dst_hbm.at[pl.ds(base, GC)], dstv, sem),
               pltpu.async_copy(src_hbm.at[pl.ds(base, GC)], srcv, sem)]
        for j in range(GC // 80):
            cps.append(pltpu.async_copy(
                slot_hbm.at[pl.ds(base + j * 80, 80)], slotv.at[j], sem))
        for cp in cps:
            cp.wait()
        cps = [pltpu.async_copy(a_hbm.at[dstv], arows, sem),
               pltpu.async_copy(b_hbm.at[srcv], brows, sem)]
        for cp in cps:
            cp.wait()

        def row(r, _):
            for k in range(F // 16):
                s = pl.ds(k * 16, 16)
                arows[r, s] = jnp.maximum(arows[r, s] + brows[r, s], 0.0)
            return 0

        lax.fori_loop(0, GC, row, 0)
        cps = [pltpu.async_copy(arows.at[pl.ds(j * 80, 80)],
                                t_hbm.at[slotv.at[j]], sem)
               for j in range(GC // 80)]
        for cp in cps:
            cp.wait()
        return 0

    lax.fori_loop(0, EPW // GC, chunk, 0)


@functools.partial(
    pl.kernel,
    out_type=jax.ShapeDtypeStruct((N_PAD, F), jnp.float32),
    mesh=_mesh,
    scratch_types=[
        pltpu.VMEM((NW, NW), jnp.int32),
        pltpu.VMEM((SUB,), jnp.int32),
        pltpu.VMEM((SUB, F), jnp.float32),
        pltpu.VMEM((NPW + 1, F), jnp.float32),
        pltpu.SemaphoreType.DMA,
    ],
    compiler_params=_sc_params,
)
def _seg_max(m_hbm, bdstl_hbm, cnt_hbm, h_hbm, cntv, dstlv, mrows, acc, sem):
    wid = _wid()
    lanes = lax.iota(jnp.int32, 16)
    neg_inf = jnp.full((16,), -jnp.inf, dtype=jnp.float32)
    pltpu.sync_copy(cnt_hbm, cntv)

    t0 = jnp.zeros((16,), jnp.int32)
    t1 = jnp.zeros((16,), jnp.int32)
    for w in range(NW):
        t0 = t0 + cntv[w, pl.ds(0, 16)]
        t1 = t1 + cntv[w, pl.ds(16, 16)]
    r80 = (t0 + 7) & ~7
    r81 = (t1 + 7) & ~7
    ex0 = jnp.cumsum(r80) - r80
    ex1 = jnp.cumsum(r81) - r81 + jnp.sum(r80)
    m0 = jnp.where(lanes == wid, 1, 0)
    m1 = jnp.where(lanes == wid - 16, 1, 0)
    bstart = jnp.sum(ex0 * m0) + jnp.sum(ex1 * m1)
    total = jnp.sum(t0 * m0) + jnp.sum(t1 * m1)

    def init_acc(r, _):
        for k in range(F // 16):
            acc[r, pl.ds(k * 16, 16)] = neg_inf
        return 0

    lax.fori_loop(0, NPW + 1, init_acc, 0)

    def chunk(c, _):
        off = _al8(bstart + c * SUB)
        cps = [pltpu.async_copy(bdstl_hbm.at[pl.ds(off, SUB)], dstlv, sem),
               pltpu.async_copy(m_hbm.at[pl.ds(off, SUB)], mrows, sem)]
        for cp in cps:
            cp.wait()
        cnt = jnp.minimum(total - c * SUB, SUB)

        def group(g, _):
            d16 = dstlv[pl.ds(g * 16, 16)]
            valid = lanes < (cnt - g * 16)
            d16s = jnp.where(valid, d16, NPW)
            for lane in range(16):
                dloc = d16s[lane]
                r = g * 16 + lane
                for k in range(F // 16):
                    s = pl.ds(k * 16, 16)
                    acc[dloc, s] = jnp.maximum(acc[dloc, s], mrows[r, s])
            return 0

        lax.fori_loop(0, (cnt + 15) // 16, group, 0)
        return 0

    lax.fori_loop(0, (total + SUB - 1) // SUB, chunk, 0)
    pltpu.sync_copy(acc.at[pl.ds(0, NPW)], h_hbm.at[pl.ds(wid * NPW, NPW)])


def _ab_kernel(h_ref, wd_ref, wb_ref, ba_ref, oa_ref, ob_ref):
    h = h_ref[...]
    oa_ref[...] = h @ wd_ref[...] + ba_ref[...]
    ob_ref[...] = h @ wb_ref[...]


def _ab_from_agg_kernel(hraw_ref, bprev_ref, wd_ref, wb_ref, ba_ref,
                        oa_ref, ob_ref):
    hr = hraw_ref[...]
    h = jnp.maximum(jnp.where(jnp.isfinite(hr), hr + bprev_ref[...], 0.0), 0.0)
    oa_ref[...] = h @ wd_ref[...] + ba_ref[...]
    ob_ref[...] = h @ wb_ref[...]


def _mm_kernel(t_ref, w_ref, o_ref):
    o_ref[...] = t_ref[...] @ w_ref[...]


def _head_kernel(hraw_ref, bprev_ref, wl_ref, bl_ref, o_ref):
    hr = hraw_ref[...]
    h = jnp.maximum(jnp.where(jnp.isfinite(hr), hr + bprev_ref[...], 0.0), 0.0)
    o_ref[...] = h @ wl_ref[...] + bl_ref[...]


def _tc_ab(h, wd, wbot, ba):
    n = h.shape[0]
    return pl.pallas_call(
        _ab_kernel,
        out_shape=(jax.ShapeDtypeStruct((n, F), jnp.float32),
                   jax.ShapeDtypeStruct((n, F), jnp.float32)),
    )(h, wd, wbot, ba.reshape(1, F))


def _tc_ab_from_agg(hraw, bprev, wd, wbot, ba):
    n = hraw.shape[0]
    return pl.pallas_call(
        _ab_from_agg_kernel,
        out_shape=(jax.ShapeDtypeStruct((n, F), jnp.float32),
                   jax.ShapeDtypeStruct((n, F), jnp.float32)),
    )(hraw, bprev.reshape(1, F), wd, wbot, ba.reshape(1, F))


def _tc_mm(t, w):
    blk = 8000
    return pl.pallas_call(
        _mm_kernel,
        grid=(E_PAD // blk,),
        in_specs=[pl.BlockSpec((blk, F), lambda i: (i, 0)),
                  pl.BlockSpec((F, F), lambda i: (0, 0))],
        out_specs=pl.BlockSpec((blk, F), lambda i: (i, 0)),
        out_shape=jax.ShapeDtypeStruct((E_PAD, F), jnp.float32),
    )(t, w)


def _tc_head(hraw, bprev, wl, bl):
    n = hraw.shape[0]
    return pl.pallas_call(
        _head_kernel,
        out_shape=jax.ShapeDtypeStruct((n, 1), jnp.float32),
    )(hraw, bprev.reshape(1, F), wl, bl.reshape(1, 1))


def kernel(x, edge_index, W1a, b1a, W2a, b2a, W1b, b1b, W2b, b2b, Wl, bl):
    src = edge_index[0]
    dst = edge_index[1]

    d = x.shape[1]
    wd1, wbot1 = W1a[:d] - W1a[d:], W1a[d:]
    wd2, wbot2 = W1b[:F] - W1b[F:], W1b[F:]

    counts = _hist(dst)
    slots, bdstl = _binner(dst, counts)

    a1, b1 = _tc_ab(x, wd1, wbot1, b1a)
    t1 = _edge_gather(a1, b1, src, dst, slots)
    m1 = _tc_mm(t1, W2a)
    h1 = _seg_max(m1, bdstl, counts)

    a2, b2 = _tc_ab_from_agg(h1, b2a, wd2, wbot2, b1b)
    t2 = _edge_gather(a2, b2, src, dst, slots)
    m2 = _tc_mm(t2, W2b)
    h2 = _seg_max(m2, bdstl, counts)

    out = _tc_head(h2, b2b, Wl, bl)
    return out[:N_NODES, 0]
